# baseline (speedup 1.0000x reference)
import functools

import jax
import jax.numpy as jnp
import numpy as np
from jax import lax
from jax.experimental import pallas as pl
from jax.experimental.pallas import tpu as pltpu
from jax.experimental.pallas import tpu_sc as plsc

N = 16384
COLS = 256
CG = 64
NG = COLS // CG
NSC = 2
NTILE = 16
GPC = NG // NSC
CHUNK = 256
SCH = 8
ROWS_PER_TILE = N // NTILE

_PERM = np.empty((CG,), np.int32)
for _h in range(2):
    for _i in range(16):
        _PERM[32 * _h + 2 * _i] = 32 * _h + _i
        _PERM[32 * _h + 2 * _i + 1] = 32 * _h + 16 + _i


def _scale_chunk(gb16, gf32, ibig, u):
    def scale_body(blk, carry):
        vv16i = ibig[u, pl.ds(2 * CHUNK + blk * 16, 16)]
        vv16 = plsc.bitcast(vv16i, jnp.float32)
        vvs = [jnp.full((16,), vv16[v], jnp.float32) for v in range(16)]
        quads = []
        for v in range(16):
            i = blk * 16 + v
            x0 = gb16[i, pl.ds(0, 32)]
            x1 = gb16[i, pl.ds(32, 32)]
            a0, b0 = plsc.unpack(x0, format=plsc.PackFormat.INTERLEAVED)
            a1, b1 = plsc.unpack(x1, format=plsc.PackFormat.INTERLEAVED)
            quads.append((a0, b0, a1, b1))
        for v in range(16):
            i = blk * 16 + v
            for q in range(4):
                gf32[i, pl.ds(q * 16, 16)] = quads[v][q] * vvs[v]
        return carry

    lax.fori_loop(0, CHUNK // 16, scale_body, 0)


def _sm_body(idx3_hbm, table_hbm, zinit_hbm, out_hbm,
             acc, ibig, gb16_0, gb16_1, gf32_0, gf32_1,
             isem, gsem0, gsem1, ssem0, ssem1, *, nsch, ntile_sch):
    c = lax.axis_index("c")
    s = lax.axis_index("s")
    base = s * ntile_sch
    gb16s = (gb16_0, gb16_1)
    gf32s = (gf32_0, gf32_1)
    gsems = (gsem0, gsem1)
    ssems = (ssem0, ssem1)

    for j in range(GPC):
        g = c * GPC + j
        off = (g * N).astype(jnp.int32)
        offv = jnp.full((16,), off, jnp.int32)

        pltpu.sync_copy(zinit_hbm, acc.at[pl.ds(s * ROWS_PER_TILE, ROWS_PER_TILE)])
        plsc.subcore_barrier()

        def sch_body(t, carry):
            row = base + t * SCH
            pltpu.async_copy(idx3_hbm.at[pl.ds(row, SCH)], ibig, isem).wait()

            gds = [None] * SCH
            sds = [None] * SCH

            def process(k):
                gds[k].wait()
                if k >= 2:
                    sds[k - 2].wait()
                _scale_chunk(gb16s[k % 2], gf32s[k % 2], ibig, k)
                sds[k] = pltpu.async_copy(
                    gf32s[k % 2], acc.at[ibig.at[k, pl.ds(CHUNK, CHUNK)]],
                    ssems[k % 2], add=True)

            for u in range(SCH):
                for q in range(CHUNK // 16):
                    sl = pl.ds(q * 16, 16)
                    ibig[u, sl] = ibig[u, sl] + offv
                gds[u] = pltpu.async_copy(
                    table_hbm.at[ibig.at[u, pl.ds(0, CHUNK)]],
                    gb16s[u % 2], gsems[u % 2])
                if u >= 1:
                    process(u - 1)
            process(SCH - 1)
            sds[SCH - 2].wait()
            sds[SCH - 1].wait()
            return carry

        lax.fori_loop(0, nsch, sch_body, 0)
        plsc.subcore_barrier()
        pltpu.sync_copy(acc.at[pl.ds(s * ROWS_PER_TILE, ROWS_PER_TILE)],
                        out_hbm.at[pl.ds(off + s * ROWS_PER_TILE, ROWS_PER_TILE)])


def kernel(indices, values, shape, b):
    del shape
    nnz = values.shape[0]
    step = NTILE * CHUNK * SCH
    nnz_pad = ((nnz + step - 1) // step) * step
    pad = nnz_pad - nnz
    nrow = nnz_pad // CHUNK
    rows = jnp.concatenate([indices[0], jnp.zeros((pad,), jnp.int32)])
    cols = jnp.concatenate([indices[1], jnp.zeros((pad,), jnp.int32)])
    vals = jnp.concatenate([values, jnp.zeros((pad,), jnp.float32)])
    idx3 = jnp.concatenate(
        [cols.reshape(nrow, CHUNK), rows.reshape(nrow, CHUNK),
         lax.bitcast_convert_type(vals.reshape(nrow, CHUNK), jnp.int32)],
        axis=1)
    table = (b.reshape(N, NG, CG).transpose(1, 0, 2).reshape(NG * N, CG)
             [:, _PERM].astype(jnp.bfloat16))
    zinit = jnp.zeros((ROWS_PER_TILE, CG), jnp.float32)

    ntile_sch = nrow // NTILE
    nsch = ntile_sch // SCH

    mesh = plsc.VectorSubcoreMesh(core_axis_name="c", subcore_axis_name="s")
    body = functools.partial(_sm_body, nsch=nsch, ntile_sch=ntile_sch)
    out_flat = pl.kernel(
        body,
        out_type=jax.ShapeDtypeStruct((NG * N, CG), jnp.float32),
        mesh=mesh,
        compiler_params=pltpu.CompilerParams(use_tc_tiling_on_sc=False,
                                             needs_layout_passes=False),
        scratch_types=[
            pltpu.VMEM_SHARED((N, CG), jnp.float32),
            pltpu.VMEM((SCH, 3 * CHUNK), jnp.int32),
            pltpu.VMEM((CHUNK, CG), jnp.bfloat16),
            pltpu.VMEM((CHUNK, CG), jnp.bfloat16),
            pltpu.VMEM((CHUNK, CG), jnp.float32),
            pltpu.VMEM((CHUNK, CG), jnp.float32),
            pltpu.SemaphoreType.DMA,
            pltpu.SemaphoreType.DMA,
            pltpu.SemaphoreType.DMA,
            pltpu.SemaphoreType.DMA,
            pltpu.SemaphoreType.DMA,
        ],
    )(idx3, table, zinit)
    return out_flat.reshape(NG, N, CG).transpose(1, 0, 2).reshape(N, COLS)


# speedup vs baseline: 1.1178x; 1.1178x over previous
import functools

import jax
import jax.numpy as jnp
import numpy as np
from jax import lax
from jax.experimental import pallas as pl
from jax.experimental.pallas import tpu as pltpu
from jax.experimental.pallas import tpu_sc as plsc

N = 16384
COLS = 256
CG = 64
NG = COLS // CG
NSC = 2
NTILE = 16
GPC = NG // NSC
CHUNK = 256
SCH = 16
ROWS_PER_TILE = N // NTILE

_PERM = np.empty((CG,), np.int32)
for _h in range(2):
    for _i in range(16):
        _PERM[32 * _h + 2 * _i] = 32 * _h + _i
        _PERM[32 * _h + 2 * _i + 1] = 32 * _h + 16 + _i


def _scale_chunk(gb16, gf32, vbig, u):
    def scale_body(blk, carry):
        vv16 = vbig[u, pl.ds(blk * 16, 16)]
        vvs = [jnp.full((16,), vv16[v], jnp.float32) for v in range(16)]
        quads = []
        for v in range(16):
            i = blk * 16 + v
            x0 = gb16[i, pl.ds(0, 32)]
            x1 = gb16[i, pl.ds(32, 32)]
            a0, b0 = plsc.unpack(x0, format=plsc.PackFormat.INTERLEAVED)
            a1, b1 = plsc.unpack(x1, format=plsc.PackFormat.INTERLEAVED)
            quads.append((a0, b0, a1, b1))
        for v in range(16):
            i = blk * 16 + v
            for q in range(4):
                gf32[i, pl.ds(q * 16, 16)] = quads[v][q] * vvs[v]
        return carry

    lax.fori_loop(0, CHUNK // 16, scale_body, 0)


def _sm_body(rows_hbm, cols_hbm, vals_hbm, table_hbm, zinit_hbm, out_hbm,
             acc, rbig, cbig, vbig, gb16_0, gb16_1, gf32_0, gf32_1,
             isem, gsem0, gsem1, ssem0, ssem1, *, nsch, ntile_sch):
    c = lax.axis_index("c")
    s = lax.axis_index("s")
    base = s * ntile_sch
    gb16s = (gb16_0, gb16_1)
    gf32s = (gf32_0, gf32_1)
    gsems = (gsem0, gsem1)
    ssems = (ssem0, ssem1)

    for j in range(GPC):
        g = c * GPC + j
        off = (g * N).astype(jnp.int32)
        offv = jnp.full((16,), off, jnp.int32)

        pltpu.sync_copy(zinit_hbm, acc.at[pl.ds(s * ROWS_PER_TILE, ROWS_PER_TILE)])
        plsc.subcore_barrier()

        def sch_body(t, carry):
            row = base + t * SCH
            ir = pltpu.async_copy(rows_hbm.at[pl.ds(row, SCH)], rbig, isem)
            ic = pltpu.async_copy(cols_hbm.at[pl.ds(row, SCH)], cbig, isem)
            iv = pltpu.async_copy(vals_hbm.at[pl.ds(row, SCH)], vbig, isem)
            ir.wait()
            ic.wait()
            iv.wait()

            gds = [None] * SCH
            sds = [None] * SCH

            def process(k):
                gds[k].wait()
                if k >= 2:
                    sds[k - 2].wait()
                _scale_chunk(gb16s[k % 2], gf32s[k % 2], vbig, k)
                sds[k] = pltpu.async_copy(
                    gf32s[k % 2], acc.at[rbig.at[k]], ssems[k % 2], add=True)

            for u in range(SCH):
                for q in range(CHUNK // 16):
                    sl = pl.ds(q * 16, 16)
                    cbig[u, sl] = cbig[u, sl] + offv
                gds[u] = pltpu.async_copy(
                    table_hbm.at[cbig.at[u]], gb16s[u % 2], gsems[u % 2])
                if u >= 1:
                    process(u - 1)
            process(SCH - 1)
            sds[SCH - 2].wait()
            sds[SCH - 1].wait()
            return carry

        lax.fori_loop(0, nsch, sch_body, 0)
        plsc.subcore_barrier()
        pltpu.sync_copy(acc.at[pl.ds(s * ROWS_PER_TILE, ROWS_PER_TILE)],
                        out_hbm.at[pl.ds(off + s * ROWS_PER_TILE, ROWS_PER_TILE)])


def kernel(indices, values, shape, b):
    del shape
    nnz = values.shape[0]
    step = NTILE * CHUNK * SCH
    nnz_pad = ((nnz + step - 1) // step) * step
    pad = nnz_pad - nnz
    nrow = nnz_pad // CHUNK
    rows = jnp.concatenate([indices[0], jnp.zeros((pad,), jnp.int32)])
    cols = jnp.concatenate([indices[1], jnp.zeros((pad,), jnp.int32)])
    vals = jnp.concatenate([values, jnp.zeros((pad,), jnp.float32)])
    rows2 = rows.reshape(nrow, CHUNK)
    cols2 = cols.reshape(nrow, CHUNK)
    vals2 = vals.reshape(nrow, CHUNK)
    table = (b.reshape(N, NG, CG).transpose(1, 0, 2).reshape(NG * N, CG)
             [:, _PERM].astype(jnp.bfloat16))
    zinit = jnp.zeros((ROWS_PER_TILE, CG), jnp.float32)

    ntile_sch = nrow // NTILE
    nsch = ntile_sch // SCH

    mesh = plsc.VectorSubcoreMesh(core_axis_name="c", subcore_axis_name="s")
    body = functools.partial(_sm_body, nsch=nsch, ntile_sch=ntile_sch)
    out_flat = pl.kernel(
        body,
        out_type=jax.ShapeDtypeStruct((NG * N, CG), jnp.float32),
        mesh=mesh,
        compiler_params=pltpu.CompilerParams(use_tc_tiling_on_sc=False,
                                             needs_layout_passes=False),
        scratch_types=[
            pltpu.VMEM_SHARED((N, CG), jnp.float32),
            pltpu.VMEM((SCH, CHUNK), jnp.int32),
            pltpu.VMEM((SCH, CHUNK), jnp.int32),
            pltpu.VMEM((SCH, CHUNK), jnp.float32),
            pltpu.VMEM((CHUNK, CG), jnp.bfloat16),
            pltpu.VMEM((CHUNK, CG), jnp.bfloat16),
            pltpu.VMEM((CHUNK, CG), jnp.float32),
            pltpu.VMEM((CHUNK, CG), jnp.float32),
            pltpu.SemaphoreType.DMA,
            pltpu.SemaphoreType.DMA,
            pltpu.SemaphoreType.DMA,
            pltpu.SemaphoreType.DMA,
            pltpu.SemaphoreType.DMA,
        ],
    )(rows2, cols2, vals2, table, zinit)
    return out_flat.reshape(NG, N, CG).transpose(1, 0, 2).reshape(N, COLS)
